# baseline (speedup 1.0000x reference)
import jax
import jax.numpy as jnp
from jax.experimental import pallas as pl
from jax.experimental.pallas import tpu as pltpu

_BM = 400
_BK = 2048


def _fused_kernel(x_ref, w_ref, adj_ref, b_ref, o_ref, h_ref, *, n):
    m = pl.program_id(0)
    k = pl.program_id(1)

    @pl.when(m == 0)
    def _compute_h_slice():
        h = jnp.dot(x_ref[...], w_ref[...],
                    preferred_element_type=jnp.float32)
        rows = jax.lax.broadcasted_iota(jnp.int32, h.shape, 0)
        h = jnp.where(k * _BK + rows < n, h, 0.0)
        h_ref[k] = h.astype(jnp.bfloat16)

    @pl.when(k == 0)
    def _init_out():
        o_ref[...] = jnp.broadcast_to(b_ref[...], o_ref.shape)

    adjb = adj_ref[...].astype(jnp.bfloat16)
    cols = jax.lax.broadcasted_iota(jnp.int32, adjb.shape, 1)
    adjb = jnp.where(k * _BK + cols < n, adjb, jnp.bfloat16(0))
    o_ref[...] += jnp.dot(adjb, h_ref[k],
                          preferred_element_type=jnp.float32)


def kernel(input, adj, W, b):
    n, d_in = input.shape
    d_out = W.shape[1]
    gm = n // _BM
    gk = pl.cdiv(n, _BK)

    import functools
    body = functools.partial(_fused_kernel, n=n)

    return pl.pallas_call(
        body,
        grid=(gm, gk),
        in_specs=[
            pl.BlockSpec((_BK, d_in), lambda m, k: (k, 0)),
            pl.BlockSpec((d_in, d_out), lambda m, k: (0, 0)),
            pl.BlockSpec((_BM, _BK), lambda m, k: (m, k)),
            pl.BlockSpec((1, d_out), lambda m, k: (0, 0)),
        ],
        out_specs=pl.BlockSpec((_BM, d_out), lambda m, k: (m, 0)),
        out_shape=jax.ShapeDtypeStruct((n, d_out), jnp.float32),
        scratch_shapes=[
            pltpu.VMEM((gk, _BK, d_out), jnp.bfloat16),
        ],
        compiler_params=pltpu.CompilerParams(
            dimension_semantics=("parallel", "arbitrary"),
        ),
    )(input, W, adj, b)


# speedup vs baseline: 1.8268x; 1.8268x over previous
import jax
import jax.numpy as jnp
from jax import lax
from jax.experimental import pallas as pl
from jax.experimental.pallas import tpu as pltpu


def _xw_kernel(x_ref, w_ref, o_ref):
    h = jnp.dot(x_ref[...], w_ref[...], preferred_element_type=jnp.float32)
    o_ref[...] = h.astype(jnp.bfloat16)


def _agg_kernel(adj_ref, h_ref, b_ref, o_ref):
    acc = lax.dot_general(
        adj_ref[...], h_ref[...],
        dimension_numbers=(((1,), (0,)), ((), ())),
        preferred_element_type=jnp.float32,
    )
    o_ref[...] = acc + b_ref[...]


def kernel(input, adj, W, b):
    n, d_in = input.shape
    d_out = W.shape[1]

    bm1 = 2000
    h = pl.pallas_call(
        _xw_kernel,
        grid=(n // bm1,),
        in_specs=[
            pl.BlockSpec((bm1, d_in), lambda i: (i, 0)),
            pl.BlockSpec((d_in, d_out), lambda i: (0, 0)),
        ],
        out_specs=pl.BlockSpec((bm1, d_out), lambda i: (i, 0)),
        out_shape=jax.ShapeDtypeStruct((n, d_out), jnp.bfloat16),
        compiler_params=pltpu.CompilerParams(
            dimension_semantics=("parallel",),
        ),
    )(input, W)

    bm = 400
    out = pl.pallas_call(
        _agg_kernel,
        grid=(n // bm,),
        in_specs=[
            pl.BlockSpec((bm, n), lambda i: (i, 0)),
            pl.BlockSpec((n, d_out), lambda i: (0, 0)),
            pl.BlockSpec((1, d_out), lambda i: (0, 0)),
        ],
        out_specs=pl.BlockSpec((bm, d_out), lambda i: (i, 0)),
        out_shape=jax.ShapeDtypeStruct((n, d_out), jnp.float32),
        compiler_params=pltpu.CompilerParams(
            dimension_semantics=("parallel",),
        ),
    )(adj, h, b)
    return out


# speedup vs baseline: 1.8607x; 1.0186x over previous
import jax
import jax.numpy as jnp
from jax.experimental import pallas as pl
from jax.experimental.pallas import tpu as pltpu


def _xw_kernel(x_ref, w_ref, o_ref):
    h = jnp.dot(x_ref[...], w_ref[...], preferred_element_type=jnp.float32)
    o_ref[...] = h.astype(jnp.bfloat16)


def _agg_kernel(adj_ref, h_ref, b_ref, o_ref):
    acc = jnp.dot(adj_ref[...].astype(jnp.bfloat16), h_ref[...],
                  preferred_element_type=jnp.float32)
    o_ref[...] = acc + b_ref[...]


def kernel(input, adj, W, b):
    n, d_in = input.shape
    d_out = W.shape[1]

    bm1 = 2000
    h = pl.pallas_call(
        _xw_kernel,
        grid=(n // bm1,),
        in_specs=[
            pl.BlockSpec((bm1, d_in), lambda i: (i, 0)),
            pl.BlockSpec((d_in, d_out), lambda i: (0, 0)),
        ],
        out_specs=pl.BlockSpec((bm1, d_out), lambda i: (i, 0)),
        out_shape=jax.ShapeDtypeStruct((n, d_out), jnp.bfloat16),
        compiler_params=pltpu.CompilerParams(
            dimension_semantics=("parallel",),
        ),
    )(input, W)

    bm = 512
    out = pl.pallas_call(
        _agg_kernel,
        grid=(pl.cdiv(n, bm),),
        in_specs=[
            pl.BlockSpec((bm, n), lambda i: (i, 0)),
            pl.BlockSpec((n, d_out), lambda i: (0, 0)),
            pl.BlockSpec((1, d_out), lambda i: (0, 0)),
        ],
        out_specs=pl.BlockSpec((bm, d_out), lambda i: (i, 0)),
        out_shape=jax.ShapeDtypeStruct((n, d_out), jnp.float32),
        compiler_params=pltpu.CompilerParams(
            dimension_semantics=("parallel",),
        ),
    )(adj, h, b)
    return out


# speedup vs baseline: 1.9325x; 1.0386x over previous
import functools

import jax
import jax.numpy as jnp
from jax.experimental import pallas as pl
from jax.experimental.pallas import tpu as pltpu

_BM = 480
_BX = 1000


def _fused_kernel(x_ref, w_ref, adj_ref, b_ref, o_ref, h_ref, *, gx):
    i = pl.program_id(0)

    @pl.when(i < gx)
    def _build_h_slice():
        h = jnp.dot(x_ref[...], w_ref[...],
                    preferred_element_type=jnp.float32)
        h_ref[pl.ds(i * _BX, _BX), :] = h.astype(jnp.bfloat16)

    @pl.when(i >= gx)
    def _aggregate_band():
        acc = jnp.dot(adj_ref[...].astype(jnp.bfloat16), h_ref[...],
                      preferred_element_type=jnp.float32)
        o_ref[...] = acc + b_ref[...]


def kernel(input, adj, W, b):
    n, d_in = input.shape
    d_out = W.shape[1]
    gx = n // _BX
    gm = pl.cdiv(n, _BM)

    body = functools.partial(_fused_kernel, gx=gx)
    last_x = gx - 1

    return pl.pallas_call(
        body,
        grid=(gx + gm,),
        in_specs=[
            pl.BlockSpec((_BX, d_in), lambda i: (jnp.minimum(i, last_x), 0)),
            pl.BlockSpec((d_in, d_out), lambda i: (0, 0)),
            pl.BlockSpec((_BM, n), lambda i: (jnp.maximum(i - gx, 0), 0)),
            pl.BlockSpec((1, d_out), lambda i: (0, 0)),
        ],
        out_specs=pl.BlockSpec(
            (_BM, d_out), lambda i: (jnp.maximum(i - gx, 0), 0)),
        out_shape=jax.ShapeDtypeStruct((n, d_out), jnp.float32),
        scratch_shapes=[
            pltpu.VMEM((n, d_out), jnp.bfloat16),
        ],
        compiler_params=pltpu.CompilerParams(
            dimension_semantics=("arbitrary",),
        ),
    )(input, W, adj, b)


# speedup vs baseline: 1.9347x; 1.0011x over previous
import functools

import jax
import jax.numpy as jnp
from jax.experimental import pallas as pl
from jax.experimental.pallas import tpu as pltpu

_BM = 480
_BX = 1000


def _fused_kernel(x_ref, w_ref, adj_ref, b_ref, o_ref, h_ref, *, gx):
    i = pl.program_id(0)

    @pl.when(i < gx)
    def _build_h_slice():
        h = jnp.dot(x_ref[...], w_ref[...],
                    preferred_element_type=jnp.float32)
        h_ref[pl.ds(i * _BX, _BX), :] = h.astype(jnp.bfloat16)

    @pl.when(i >= gx)
    def _aggregate_band():
        acc = jax.lax.dot_general(
            adj_ref[...], h_ref[...],
            dimension_numbers=(((1,), (0,)), ((), ())),
            preferred_element_type=jnp.float32,
        )
        o_ref[...] = acc + b_ref[...]


def kernel(input, adj, W, b):
    n, d_in = input.shape
    d_out = W.shape[1]
    gx = n // _BX
    gm = pl.cdiv(n, _BM)

    body = functools.partial(_fused_kernel, gx=gx)
    last_x = gx - 1

    return pl.pallas_call(
        body,
        grid=(gx + gm,),
        in_specs=[
            pl.BlockSpec((_BX, d_in), lambda i: (jnp.minimum(i, last_x), 0)),
            pl.BlockSpec((d_in, d_out), lambda i: (0, 0)),
            pl.BlockSpec((_BM, n), lambda i: (jnp.maximum(i - gx, 0), 0)),
            pl.BlockSpec((1, d_out), lambda i: (0, 0)),
        ],
        out_specs=pl.BlockSpec(
            (_BM, d_out), lambda i: (jnp.maximum(i - gx, 0), 0)),
        out_shape=jax.ShapeDtypeStruct((n, d_out), jnp.float32),
        scratch_shapes=[
            pltpu.VMEM((n, d_out), jnp.bfloat16),
        ],
        compiler_params=pltpu.CompilerParams(
            dimension_semantics=("arbitrary",),
        ),
    )(input, W, adj, b)
